# Initial kernel scaffold; baseline (speedup 1.0000x reference)
import jax
import jax.numpy as jnp
from jax.experimental import pallas as pl


def kernel(hidden_states, expert_affinities, expert_index, gate_up_proj, down_proj):
    raise NotImplementedError("write your pallas kernel here")


# speedup vs baseline: 1.0886x; 1.0886x over previous
import functools

import jax
import jax.numpy as jnp
from jax.experimental import pallas as pl

_E = 8
_TOP_K = 2
_T = 32
_H = 2048
_I = 4096
_TS = 512
_NI = _I // _TS


def _routing_weights(idx, aff):
    t, e = aff.shape
    erange = jax.lax.broadcasted_iota(jnp.int32, (t, e), 1)
    chosen = jnp.zeros((t, e), dtype=jnp.bool_)
    for k in range(idx.shape[1]):
        chosen = chosen | (idx[:, k][:, None] == erange)
    aff_m = jnp.where(chosen, aff, 0.0)
    denom = jnp.maximum(jnp.sum(jnp.abs(aff_m), axis=1, keepdims=True), 1e-12)
    return aff_m / denom


def _mlp_kernel(idx_ref, aff_ref, x_ref, gate_ref, up_ref, down_ref, out_ref):
    e = pl.program_id(0)
    i = pl.program_id(1)

    w = _routing_weights(idx_ref[:, :], aff_ref[:, :])
    ecol = jax.lax.broadcasted_iota(jnp.int32, w.shape, 1)
    we = jnp.sum(jnp.where(ecol == e, w, 0.0), axis=1, keepdims=True)

    x = x_ref[:, :]
    g = jnp.dot(x, gate_ref[0], preferred_element_type=jnp.float32)
    u = jnp.dot(x, up_ref[0], preferred_element_type=jnp.float32)
    inter = (g * jax.nn.sigmoid(g)) * u * we
    contrib = jnp.dot(inter, down_ref[0], preferred_element_type=jnp.float32)

    @pl.when((e == 0) & (i == 0))
    def _init():
        out_ref[:, :] = jnp.zeros_like(out_ref)

    out_ref[:, :] += contrib


@functools.partial(jax.jit, static_argnames=())
def kernel(hidden_states, expert_affinities, expert_index, gate_up_proj, down_proj):
    idx = expert_index.astype(jnp.int32)
    grid = (_E, _NI)
    return pl.pallas_call(
        _mlp_kernel,
        grid=grid,
        in_specs=[
            pl.BlockSpec((_T, _TOP_K), lambda e, i: (0, 0)),
            pl.BlockSpec((_T, _E), lambda e, i: (0, 0)),
            pl.BlockSpec((_T, _H), lambda e, i: (0, 0)),
            pl.BlockSpec((1, _H, _TS), lambda e, i: (e, 0, i)),
            pl.BlockSpec((1, _H, _TS), lambda e, i: (e, 0, _NI + i)),
            pl.BlockSpec((1, _TS, _H), lambda e, i: (e, i, 0)),
        ],
        out_specs=pl.BlockSpec((_T, _H), lambda e, i: (0, 0)),
        out_shape=jax.ShapeDtypeStruct((_T, _H), jnp.float32),
    )(idx, expert_affinities, hidden_states, gate_up_proj, gate_up_proj, down_proj)
